# baseline (speedup 1.0000x reference)
import functools

import jax
import jax.numpy as jnp
from jax import lax
from jax.experimental import pallas as pl
from jax.experimental.pallas import tpu as pltpu
from jax.experimental.pallas import tpu_sc as plsc

N_MOL, E_MOL = 10000, 320000
N_PROT, E_PROT = 1000, 32000
B = 64
H, NH = 64, 4
HD = H // NH
NC, NS = 2, 16
NW = NC * NS
F32 = jnp.float32


NT = N_MOL + N_PROT
CHM, CHP = 80, 40
EWM, EWP = E_MOL // NW, E_PROT // NW
CM, CP = EWM // CHM, EWP // CHP
ZCH = 200
ZC = NT // ZCH
ZITER = (ZC + NS - 1) // NS


def _make_agg_both():
    mesh = plsc.VectorSubcoreMesh(core_axis_name="c", subcore_axis_name="s")

    @functools.partial(
        pl.kernel,
        out_type=jax.ShapeDtypeStruct((NC, NT, H), F32),
        mesh=mesh,
        scratch_types=[
            pltpu.VMEM((EWM,), jnp.int32),
            pltpu.VMEM((EWM + 16,), F32),
            pltpu.VMEM((EWM + 16,), F32),
            pltpu.VMEM((EWP,), jnp.int32),
            pltpu.VMEM((EWP + 16,), F32),
            pltpu.VMEM((EWP + 16,), F32),
            pltpu.VMEM((CHM,), jnp.int32),
            pltpu.VMEM((CHP,), jnp.int32),
            pltpu.VMEM((CHM, H), F32),
            pltpu.VMEM((CHM, H), F32),
            pltpu.VMEM((CHP, H), F32),
            pltpu.VMEM((CHP, H), F32),
            pltpu.VMEM((6, H), F32),
            pltpu.VMEM((ZCH, H), F32),
            pltpu.VMEM_SHARED((NT, H), F32),
            pltpu.SemaphoreType.DMA,
            pltpu.SemaphoreType.DMA,
        ],
        compiler_params=pltpu.CompilerParams(use_tc_tiling_on_sc=False),
    )
    def k(xm_hbm, xp_hbm, eim_hbm, eip_hbm, atm_hbm, atp_hbm, wb_hbm,
          out_hbm, srcm_v, a0m_v, a1m_v, srcp_v, a0p_v,
          a1p_v, dcm_v, dcp_v, xgm, mgm, xgp, mgp, wb_v, stage_v, agg_sh,
          gsem, dsem):
        cid = lax.axis_index("c")
        sid = lax.axis_index("s")
        wid = cid * NS + sid

        pltpu.sync_copy(eim_hbm.at[0, pl.ds(wid * EWM, EWM)], srcm_v)
        pltpu.sync_copy(atm_hbm.at[0, pl.ds(wid * EWM, EWM)],
                        a0m_v.at[pl.ds(0, EWM)])
        pltpu.sync_copy(atm_hbm.at[1, pl.ds(wid * EWM, EWM)],
                        a1m_v.at[pl.ds(0, EWM)])
        pltpu.sync_copy(eip_hbm.at[0, pl.ds(wid * EWP, EWP)], srcp_v)
        pltpu.sync_copy(atp_hbm.at[0, pl.ds(wid * EWP, EWP)],
                        a0p_v.at[pl.ds(0, EWP)])
        pltpu.sync_copy(atp_hbm.at[1, pl.ds(wid * EWP, EWP)],
                        a1p_v.at[pl.ds(0, EWP)])
        pltpu.sync_copy(wb_hbm, wb_v)

        def zrow(r, carry):
            for i in range(4):
                stage_v[r, pl.ds(16 * i, 16)] = jnp.zeros((16,), F32)
            return carry
        lax.fori_loop(0, ZCH, zrow, 0)
        for kk in range(ZITER):
            zc = sid + kk * NS

            @pl.when(zc < ZC)
            def _():
                pltpu.sync_copy(stage_v, agg_sh.at[pl.ds(zc * ZCH, ZCH)])
        plsc.subcore_barrier()

        w0m = [wb_v[0, pl.ds(16 * i, 16)] for i in range(4)]
        w1m = [wb_v[1, pl.ds(16 * i, 16)] for i in range(4)]
        bbm = [wb_v[2, pl.ds(16 * i, 16)] for i in range(4)]
        w0p = [wb_v[3, pl.ds(16 * i, 16)] for i in range(4)]
        w1p = [wb_v[4, pl.ds(16 * i, 16)] for i in range(4)]
        bbp = [wb_v[5, pl.ds(16 * i, 16)] for i in range(4)]

        def make_chunk_body(x_hbm, ei_hbm, src_v, a0_v, a1_v, dc_v, xg, mg,
                            ew, ch, w0, w1, bb):
            gfull, tail = ch // 16, ch % 16

            def do_edges(jbase, gbase, n):
                va0 = a0_v[pl.ds(jbase + 16 * gbase, 16)]
                va1 = a1_v[pl.ds(jbase + 16 * gbase, 16)]
                for i in range(n):
                    e = gbase * 16 + i
                    a0 = va0[i]
                    a1 = va1[i]
                    for t in range(4):
                        v = xg[e, pl.ds(16 * t, 16)]
                        mg[e, pl.ds(16 * t, 16)] = jnp.maximum(
                            v + a0 * w0[t] + a1 * w1[t] + bb[t], 0.0)

            def chunk_body(j, carry):
                jbase = j * ch
                pltpu.make_async_copy(
                    x_hbm.at[src_v.at[pl.ds(jbase, ch)]], xg, gsem).start()
                pltpu.make_async_copy(
                    ei_hbm.at[1, pl.ds(wid * ew + jbase, ch)],
                    dc_v, dsem).start()
                pltpu.make_async_copy(
                    ei_hbm.at[1, pl.ds(wid * ew + jbase, ch)],
                    dc_v, dsem).wait()
                pltpu.make_async_copy(
                    x_hbm.at[src_v.at[pl.ds(jbase, ch)]], xg, gsem).wait()

                def grp_body(g, c2):
                    do_edges(jbase, g, 16)
                    return c2
                lax.fori_loop(0, gfull, grp_body, 0)
                if tail:
                    do_edges(jbase, gfull, tail)
                pltpu.sync_copy(mg, agg_sh.at[dc_v], add=True)
                return carry
            return chunk_body

        lax.fori_loop(0, CM, make_chunk_body(
            xm_hbm, eim_hbm, srcm_v, a0m_v, a1m_v, dcm_v, xgm, mgm,
            EWM, CHM, w0m, w1m, bbm), 0)
        lax.fori_loop(0, CP, make_chunk_body(
            xp_hbm, eip_hbm, srcp_v, a0p_v, a1p_v, dcp_v, xgp, mgp,
            EWP, CHP, w0p, w1p, bbp), 0)
        plsc.subcore_barrier()

        for kk in range(ZITER):
            zc = sid + kk * NS

            @pl.when(zc < ZC)
            def _():
                pltpu.sync_copy(agg_sh.at[pl.ds(zc * ZCH, ZCH)], stage_v)
                pltpu.sync_copy(stage_v, out_hbm.at[cid, pl.ds(zc * ZCH, ZCH)])

    return k


_agg_both = _make_agg_both()


def _dot(a, b):
    return jax.lax.dot_general(a, b, (((1,), (0,)), ((), ())),
                               preferred_element_type=F32)


def _prelude_body(mx_ref, mw_ref, mb_ref, px_ref, pw_ref, pb_ref,
                  om_ref, op_ref):
    om_ref[...] = _dot(mx_ref[...], mw_ref[...]) + mb_ref[...]
    op_ref[...] = _dot(px_ref[...], pw_ref[...]) + pb_ref[...]


def _prelude(mx, mw, mb, px, pw, pb):
    return pl.pallas_call(
        _prelude_body,
        out_shape=[jax.ShapeDtypeStruct((N_MOL, H), F32),
                   jax.ShapeDtypeStruct((N_PROT, H), F32)],
    )(mx, mw, mb, px, pw, pb)


def _gine_mlp_body(x_ref, agg_ref, w1_ref, b1_ref, w2_ref, b2_ref, o_ref):
    h = x_ref[...] + agg_ref[0] + agg_ref[1]
    h = jnp.maximum(_dot(h, w1_ref[...]) + b1_ref[...], 0.0)
    o_ref[...] = jnp.maximum(_dot(h, w2_ref[...]) + b2_ref[...], 0.0)


def _gine_mlp(x, agg, row_block, w1, b1, w2, b2):
    n = x.shape[0]
    full = lambda s: pl.BlockSpec(s, lambda i: (0,) * len(s))
    return pl.pallas_call(
        _gine_mlp_body,
        grid=(1,),
        in_specs=[full((n, H)),
                  pl.BlockSpec((NC, n, H), lambda i, rb=row_block: (0, rb, 0)),
                  full((H, H)), full((1, H)), full((H, H)), full((1, H))],
        out_specs=full((n, H)),
        out_shape=jax.ShapeDtypeStruct((n, H), F32),
    )(x, agg, w1, b1, w2, b2)


def _qkv_body(hm_ref, hp_ref,
              wqm_ref, bqm_ref, wkp_ref, bkp_ref, wvp_ref, bvp_ref,
              wqp_ref, bqp_ref, wkm_ref, bkm_ref, wvm_ref, bvm_ref,
              qm_ref, kp_ref, vp_ref, qp_ref, km_ref, vm_ref):
    hm = hm_ref[...]
    hp = hp_ref[...]
    qm_ref[...] = _dot(hm, wqm_ref[...]) + bqm_ref[...]
    kp_ref[...] = _dot(hp, wkp_ref[...]) + bkp_ref[...]
    vp_ref[...] = _dot(hp, wvp_ref[...]) + bvp_ref[...]
    qp_ref[...] = _dot(hp, wqp_ref[...]) + bqp_ref[...]
    km_ref[...] = _dot(hm, wkm_ref[...]) + bkm_ref[...]
    vm_ref[...] = _dot(hm, wvm_ref[...]) + bvm_ref[...]


def _qkv(hm, hp_pad, wqm, bqm, wkp, bkp, wvp, bvp, wqp, bqp, wkm, bkm,
         wvm, bvm):
    np_pad = hp_pad.shape[0]
    return pl.pallas_call(
        _qkv_body,
        out_shape=[jax.ShapeDtypeStruct((N_MOL, H), F32),
                   jax.ShapeDtypeStruct((np_pad, H), F32),
                   jax.ShapeDtypeStruct((np_pad, H), F32),
                   jax.ShapeDtypeStruct((np_pad, H), F32),
                   jax.ShapeDtypeStruct((N_MOL, H), F32),
                   jax.ShapeDtypeStruct((N_MOL, H), F32)],
    )(hm, hp_pad, wqm, bqm, wkp, bkp, wvp, bvp, wqp, bqp, wkm, bkm, wvm, bvm)


def _attn_body(nk_real, q_ref, k_ref, v_ref, res_ref, o_ref):
    q = q_ref[...]
    k = k_ref[...]
    v = v_ref[...]
    nk = k.shape[0]
    scale = 1.0 / (HD ** 0.5)
    need_mask = nk_real < nk
    if need_mask:
        kmask = lax.broadcasted_iota(jnp.int32, (1, nk), 1) < nk_real
    outs = []
    for h in range(NH):
        qh = q[:, h * HD:(h + 1) * HD] * scale
        kh = k[:, h * HD:(h + 1) * HD]
        s = jax.lax.dot_general(qh, kh, (((1,), (1,)), ((), ())),
                                preferred_element_type=F32)
        if need_mask:
            s = jnp.where(kmask, s, -1e30)
        m = jnp.max(s, axis=1, keepdims=True)
        e = jnp.exp(s - m)
        w = e / jnp.sum(e, axis=1, keepdims=True)
        outs.append(_dot(w, v[:, h * HD:(h + 1) * HD]))
    o_ref[...] = res_ref[...] + jnp.concatenate(outs, axis=1)


def _attn(q, kk, vv, res, bq, nk_real):
    nq = q.shape[0]
    nk = kk.shape[0]
    grid = (nq // bq,)
    qspec = pl.BlockSpec((bq, H), lambda i: (i, 0))
    kspec = pl.BlockSpec((nk, H), lambda i: (0, 0))
    return pl.pallas_call(
        functools.partial(_attn_body, nk_real),
        grid=grid,
        in_specs=[qspec, kspec, kspec, qspec],
        out_specs=qspec,
        out_shape=jax.ShapeDtypeStruct((nq, H), F32),
        compiler_params=pltpu.CompilerParams(
            dimension_semantics=("arbitrary",)),
    )(q, kk, vv, res)


def _pool_head_body(hm_ref, hp_ref, mb_ref, pb_ref,
                    w1_ref, b1_ref, w2_ref, b2_ref, o_ref):
    def seg_mean(h, batch, n):
        iota = lax.broadcasted_iota(jnp.int32, (n, B), 1)
        oh = (batch == iota).astype(F32)
        s = jax.lax.dot_general(oh, h, (((0,), (0,)), ((), ())),
                                preferred_element_type=F32)
        ones = jnp.ones((n, 1), F32)
        cnt = jax.lax.dot_general(oh, ones, (((0,), (0,)), ((), ())),
                                  preferred_element_type=F32)
        return s / jnp.maximum(cnt, 1.0)
    zm = seg_mean(hm_ref[...], mb_ref[...], N_MOL)
    zp = seg_mean(hp_ref[...], pb_ref[...], N_PROT)
    z = jnp.concatenate([zm, zp], axis=1)
    x = jnp.maximum(_dot(z, w1_ref[...]) + b1_ref[...], 0.0)
    y = _dot(x, w2_ref[...]) + b2_ref[...]
    o_ref[...] = 1.0 / (1.0 + jnp.exp(-y))


def _pool_head(hm, hp, mbatch, pbatch, w1, b1, w2, b2):
    return pl.pallas_call(
        _pool_head_body,
        out_shape=jax.ShapeDtypeStruct((B, 1), F32),
    )(hm, hp, mbatch, pbatch, w1, b1, w2, b2)


def kernel(mol_x, mol_edge_index, mol_edge_attr, mol_batch, prot_x,
           prot_edge_index, prot_edge_attr, prot_batch, mol_node_W,
           mol_node_b, prot_node_W, prot_node_b, mol_edge_W, mol_edge_b,
           prot_edge_W, prot_edge_b, mol_c1_W1, mol_c1_b1, mol_c1_W2,
           mol_c1_b2, mol_c2_W1, mol_c2_b1, mol_c2_W2, mol_c2_b2,
           prot_c1_W1, prot_c1_b1, prot_c1_W2, prot_c1_b2, prot_c2_W1,
           prot_c2_b1, prot_c2_W2, prot_c2_b2, mp_WQ, mp_bQ, mp_WK, mp_bK,
           mp_WV, mp_bV, pm_WQ, pm_bQ, pm_WK, pm_bK, pm_WV, pm_bV,
           fc1_W, fc1_b, fc2_W, fc2_b):
    r1 = lambda b: b.reshape(1, -1)

    atm = mol_edge_attr.T
    atp = prot_edge_attr.T
    eip = prot_edge_index + jnp.array([[0], [N_MOL]], jnp.int32)
    wb6 = jnp.concatenate([mol_edge_W, r1(mol_edge_b),
                           prot_edge_W, r1(prot_edge_b)], axis=0)

    hm, hp = _prelude(mol_x, mol_node_W, r1(mol_node_b),
                      prot_x, prot_node_W, r1(prot_node_b))

    layers = (
        (mol_c1_W1, mol_c1_b1, mol_c1_W2, mol_c1_b2,
         prot_c1_W1, prot_c1_b1, prot_c1_W2, prot_c1_b2),
        (mol_c2_W1, mol_c2_b1, mol_c2_W2, mol_c2_b2,
         prot_c2_W1, prot_c2_b1, prot_c2_W2, prot_c2_b2),
    )
    for mw1, mb1, mw2, mb2, pw1, pb1, pw2, pb2 in layers:
        agg = _agg_both(hm, hp, mol_edge_index, eip, atm, atp, wb6)
        hm = _gine_mlp(hm, agg, 0, mw1, r1(mb1), mw2, r1(mb2))
        hp = _gine_mlp(hp, agg, N_MOL // N_PROT, pw1, r1(pb1),
                       pw2, r1(pb2))

    hp_pad = jnp.pad(hp, ((0, 1024 - N_PROT), (0, 0)))
    qm, kp, vp, qp, km, vm = _qkv(
        hm, hp_pad, mp_WQ, r1(mp_bQ), mp_WK, r1(mp_bK), mp_WV, r1(mp_bV),
        pm_WQ, r1(pm_bQ), pm_WK, r1(pm_bK), pm_WV, r1(pm_bV))

    hm2 = _attn(qm, kp, vp, hm, 1000, N_PROT)
    hp2_pad = _attn(qp, km, vm, hp_pad, 128, N_MOL)
    hp2 = hp2_pad[:N_PROT]

    out = _pool_head(hm2, hp2, mol_batch.reshape(-1, 1),
                     prot_batch.reshape(-1, 1),
                     fc1_W, r1(fc1_b), fc2_W, r1(fc2_b))
    return out.reshape(B)
